# baseline (speedup 1.0000x reference)
import jax
import jax.numpy as jnp
from jax.experimental import pallas as pl

K = 16384
Q = 8192
QB = 1024
KC = 2048
_BIG_IDX = 2**30


def _quat_rotmat(q):
    w, x, y, z = q[:, 0], q[:, 1], q[:, 2], q[:, 3]
    r00 = 1.0 - 2.0 * (y * y + z * z)
    r01 = 2.0 * (x * y - w * z)
    r02 = 2.0 * (x * z + w * y)
    r10 = 2.0 * (x * y + w * z)
    r11 = 1.0 - 2.0 * (x * x + z * z)
    r12 = 2.0 * (y * z - w * x)
    r20 = 2.0 * (x * z - w * y)
    r21 = 2.0 * (y * z + w * x)
    r22 = 1.0 - 2.0 * (x * x + y * y)
    return jnp.stack([
        jnp.stack([r00, r01, r02], axis=-1),
        jnp.stack([r10, r11, r12], axis=-1),
        jnp.stack([r20, r21, r22], axis=-1),
    ], axis=-2)


def _coeffs(positions, scales, quaternions):
    qn = quaternions / jnp.linalg.norm(quaternions, axis=1, keepdims=True)
    R = _quat_rotmat(qn)
    inv_s2 = 1.0 / (scales * scales)
    A = jnp.einsum('kij,kj,klj->kil', R, inv_s2, R)
    b = -2.0 * jnp.einsum('kij,kj->ki', A, positions)
    c = jnp.einsum('ki,kij,kj->k', positions, A, positions)
    return jnp.concatenate([
        A[:, 0, 0, None], A[:, 1, 1, None], A[:, 2, 2, None],
        2.0 * A[:, 0, 1, None], 2.0 * A[:, 0, 2, None], 2.0 * A[:, 1, 2, None],
        b, c[:, None],
    ], axis=1)


def _body(pts_ref, cb_ref, idx_ref, prob_ref):
    p = pts_ref[...]
    p0 = p[:, 0:1]
    p1 = p[:, 1:2]
    p2 = p[:, 2:3]
    ft = jnp.concatenate(
        [p0 * p0, p1 * p1, p2 * p2, p0 * p1, p0 * p2, p1 * p2,
         p0, p1, p2, jnp.ones_like(p0)],
        axis=1,
    ).astype(jnp.bfloat16)

    def step(kc, carry):
        cur_min, cur_arg = carry
        ctc = cb_ref[pl.ds(kc * KC, KC), :]
        m = jax.lax.dot_general(
            ft, ctc, (((1,), (1,)), ((), ())),
            preferred_element_type=jnp.float32,
        )
        row_min = jnp.min(m, axis=1, keepdims=True)
        lane = jax.lax.broadcasted_iota(jnp.int32, (QB, KC), 1) + kc * KC
        row_arg = jnp.min(
            jnp.where(m == row_min, lane, jnp.int32(_BIG_IDX)),
            axis=1, keepdims=True,
        )
        better = row_min < cur_min
        return (jnp.where(better, row_min, cur_min),
                jnp.where(better, row_arg, cur_arg))

    init = (jnp.full((QB, 1), jnp.inf, dtype=jnp.float32),
            jnp.full((QB, 1), _BIG_IDX, dtype=jnp.int32))
    best_min, best_arg = jax.lax.fori_loop(0, K // KC, step, init)
    idx_ref[...] = best_arg
    prob_ref[...] = jnp.exp(-0.5 * best_min)


@jax.jit
def kernel(positions, scales, quaternions, sorted_points):
    cb = _coeffs(positions, scales, quaternions).astype(jnp.bfloat16)
    nb = Q // QB
    idx, prob = pl.pallas_call(
        _body,
        grid=(nb,),
        in_specs=[
            pl.BlockSpec((QB, 3), lambda i: (i, 0)),
            pl.BlockSpec((K, 10), lambda i: (0, 0)),
        ],
        out_specs=[
            pl.BlockSpec((QB, 1), lambda i: (i, 0)),
            pl.BlockSpec((QB, 1), lambda i: (i, 0)),
        ],
        out_shape=[
            jax.ShapeDtypeStruct((Q, 1), jnp.int32),
            jax.ShapeDtypeStruct((Q, 1), jnp.float32),
        ],
    )(sorted_points, cb)
    return idx.reshape(Q), prob.reshape(Q)


# speedup vs baseline: 1.0609x; 1.0609x over previous
import jax
import jax.numpy as jnp
from jax.experimental import pallas as pl

K = 16384
Q = 8192
QB = 1024
KC = 2048
_BIG_IDX = 2**30


def _quat_rotmat(q):
    w, x, y, z = q[:, 0], q[:, 1], q[:, 2], q[:, 3]
    r00 = 1.0 - 2.0 * (y * y + z * z)
    r01 = 2.0 * (x * y - w * z)
    r02 = 2.0 * (x * z + w * y)
    r10 = 2.0 * (x * y + w * z)
    r11 = 1.0 - 2.0 * (x * x + z * z)
    r12 = 2.0 * (y * z - w * x)
    r20 = 2.0 * (x * z - w * y)
    r21 = 2.0 * (y * z + w * x)
    r22 = 1.0 - 2.0 * (x * x + y * y)
    return jnp.stack([
        jnp.stack([r00, r01, r02], axis=-1),
        jnp.stack([r10, r11, r12], axis=-1),
        jnp.stack([r20, r21, r22], axis=-1),
    ], axis=-2)


def _coeffs(positions, scales, quaternions):
    qn = quaternions / jnp.linalg.norm(quaternions, axis=1, keepdims=True)
    R = _quat_rotmat(qn)
    inv_s2 = 1.0 / (scales * scales)
    A = jnp.einsum('kij,kj,klj->kil', R, inv_s2, R)
    b = -2.0 * jnp.einsum('kij,kj->ki', A, positions)
    c = jnp.einsum('ki,kij,kj->k', positions, A, positions)
    return jnp.concatenate([
        A[:, 0, 0, None], A[:, 1, 1, None], A[:, 2, 2, None],
        2.0 * A[:, 0, 1, None], 2.0 * A[:, 0, 2, None], 2.0 * A[:, 1, 2, None],
        b, c[:, None],
    ], axis=1)


def _body(pts_ref, cb_ref, idx_ref, prob_ref):
    p = pts_ref[...]
    p0 = p[:, 0:1]
    p1 = p[:, 1:2]
    p2 = p[:, 2:3]
    ft = jnp.concatenate(
        [p0 * p0, p1 * p1, p2 * p2, p0 * p1, p0 * p2, p1 * p2,
         p0, p1, p2, jnp.ones_like(p0)],
        axis=1,
    ).astype(jnp.bfloat16)

    acc_v = None
    acc_i = None
    for kc in range(K // KC):
        ctc = cb_ref[pl.ds(kc * KC, KC), :]
        m = jax.lax.dot_general(
            ft, ctc, (((1,), (1,)), ((), ())),
            preferred_element_type=jnp.float32,
        )
        if kc == 0:
            acc_v = m
            acc_i = jnp.zeros((QB, KC), jnp.int32)
        else:
            upd = m < acc_v
            acc_v = jnp.where(upd, m, acc_v)
            acc_i = jnp.where(upd, jnp.int32(kc), acc_i)

    row_min = jnp.min(acc_v, axis=1, keepdims=True)
    lane = jax.lax.broadcasted_iota(jnp.int32, (QB, KC), 1)
    gidx = acc_i * jnp.int32(KC) + lane
    best_arg = jnp.min(
        jnp.where(acc_v == row_min, gidx, jnp.int32(_BIG_IDX)),
        axis=1, keepdims=True,
    )
    idx_ref[...] = best_arg
    prob_ref[...] = jnp.exp(-0.5 * row_min)


@jax.jit
def kernel(positions, scales, quaternions, sorted_points):
    cb = _coeffs(positions, scales, quaternions).astype(jnp.bfloat16)
    nb = Q // QB
    idx, prob = pl.pallas_call(
        _body,
        grid=(nb,),
        in_specs=[
            pl.BlockSpec((QB, 3), lambda i: (i, 0)),
            pl.BlockSpec((K, 10), lambda i: (0, 0)),
        ],
        out_specs=[
            pl.BlockSpec((QB, 1), lambda i: (i, 0)),
            pl.BlockSpec((QB, 1), lambda i: (i, 0)),
        ],
        out_shape=[
            jax.ShapeDtypeStruct((Q, 1), jnp.int32),
            jax.ShapeDtypeStruct((Q, 1), jnp.float32),
        ],
    )(sorted_points, cb)
    return idx.reshape(Q), prob.reshape(Q)
